# Initial kernel scaffold; baseline (speedup 1.0000x reference)
import jax
import jax.numpy as jnp
from jax.experimental import pallas as pl


def kernel(coords_lig, h_lig, orig_coords_lig, orig_h_lig, edge_feat_lig, coords_rec, h_rec, orig_coords_rec, orig_h_rec, edge_feat_rec, params, edge_index_lig, edge_index_rec):
    raise NotImplementedError("write your pallas kernel here")


# speedup vs baseline: 6.7554x; 6.7554x over previous
import functools

import jax
import jax.numpy as jnp
from jax import lax
from jax.experimental import pallas as pl
from jax.experimental.pallas import tpu as pltpu
from jax.experimental.pallas import tpu_sc as plsc

N = 10000
E = 320000
D = 64
NC, NS = 2, 16
NW = NC * NS
EW = E // NW
CH = 80
NCHUNK = EW // CH
LANES = 16

_SIGMAS = [1.5 ** i for i in range(15)]


def _prep_body(h_ref, wa_ref, wb_ref, u_ref, v_ref):
    h = h_ref[...]
    u_ref[...] = jnp.dot(h, wa_ref[...], preferred_element_type=jnp.float32)
    v_ref[...] = jnp.dot(h, wb_ref[...], preferred_element_type=jnp.float32)


def _prep(h, wa, wb):
    blk = 1000
    return pl.pallas_call(
        _prep_body,
        grid=(N // blk,),
        in_specs=[
            pl.BlockSpec((blk, D), lambda i: (i, 0)),
            pl.BlockSpec((D, D), lambda i: (0, 0)),
            pl.BlockSpec((D, D), lambda i: (0, 0)),
        ],
        out_specs=[
            pl.BlockSpec((blk, D), lambda i: (i, 0)),
            pl.BlockSpec((blk, D), lambda i: (i, 0)),
        ],
        out_shape=[
            jax.ShapeDtypeStruct((N, D), jnp.float32),
            jax.ShapeDtypeStruct((N, D), jnp.float32),
        ],
    )(h, wa, wb)


def _gather_branch(u_hbm, v_hbm, cflat_hbm, src2_hbm, dst2_hbm, w_hbm,
                   xr4_hbm, s2d, d2d, ctab, us, xr4, sem, wid):
    pltpu.sync_copy(src2_hbm.at[wid], s2d)
    pltpu.sync_copy(dst2_hbm.at[wid], d2d)
    pltpu.sync_copy(cflat_hbm, ctab)

    def chunk(j, carry):
        off = wid * EW + j * CH
        cp_u = pltpu.async_copy(u_hbm.at[s2d.at[j]], us, sem)
        cp_u.wait()
        cp_v = pltpu.async_copy(v_hbm.at[d2d.at[j]], us, sem, add=True)

        def grp(i, c):
            s16 = s2d[j, pl.ds(i * LANES, LANES)]
            d16 = d2d[j, pl.ds(i * LANES, LANES)]
            row = jnp.arange(LANES, dtype=jnp.int32) + i * LANES
            acc = jnp.zeros((LANES,), jnp.float32)
            for k in range(3):
                a = plsc.load_gather(ctab, [s16 * 3 + k])
                b = plsc.load_gather(ctab, [d16 * 3 + k])
                r = a - b
                plsc.store_scatter(xr4, [row, jnp.full((LANES,), k, jnp.int32)], r)
                acc = acc + r * r
            plsc.store_scatter(xr4, [row, jnp.full((LANES,), 3, jnp.int32)], acc)
            return c

        lax.fori_loop(0, CH // LANES, grp, 0)
        cp_v.wait()
        pltpu.sync_copy(us, w_hbm.at[pl.ds(off, CH)])
        pltpu.sync_copy(xr4, xr4_hbm.at[pl.ds(off, CH)])
        return carry

    lax.fori_loop(0, NCHUNK, chunk, 0)


def _sc_gather(u_l, v_l, cflat_l, src2_l, dst2_l,
               u_r, v_r, cflat_r, src2_r, dst2_r):
    mesh = plsc.VectorSubcoreMesh(core_axis_name="c", subcore_axis_name="s", num_cores=NC, num_subcores=NS)

    @functools.partial(
        pl.kernel,
        out_type=[
            jax.ShapeDtypeStruct((E, D), jnp.float32),
            jax.ShapeDtypeStruct((E, 4), jnp.float32),
            jax.ShapeDtypeStruct((E, D), jnp.float32),
            jax.ShapeDtypeStruct((E, 4), jnp.float32),
        ],
        mesh=mesh,
        compiler_params=pltpu.CompilerParams(needs_layout_passes=False, use_tc_tiling_on_sc=False),
        scratch_types=[
            pltpu.VMEM((NCHUNK, CH), jnp.int32),
            pltpu.VMEM((NCHUNK, CH), jnp.int32),
            pltpu.VMEM((3 * N,), jnp.float32),
            pltpu.VMEM((CH, D), jnp.float32),
            pltpu.VMEM((CH, 4), jnp.float32),
            pltpu.SemaphoreType.DMA,
        ],
    )
    def k(u_l, v_l, cflat_l, src2_l, dst2_l, u_r, v_r, cflat_r, dst2r_src,
          dst2_r, w_l, xr4_l, w_r, xr4_r, s2d, d2d, ctab, us, xr4, sem):
        wid = lax.axis_index("s") * NC + lax.axis_index("c")
        _gather_branch(u_l, v_l, cflat_l, src2_l, dst2_l, w_l, xr4_l,
                       s2d, d2d, ctab, us, xr4, sem, wid)
        _gather_branch(u_r, v_r, cflat_r, dst2r_src, dst2_r, w_r, xr4_r,
                       s2d, d2d, ctab, us, xr4, sem, wid)

    return k(u_l, v_l, cflat_l, src2_l, dst2_l,
             u_r, v_r, cflat_r, src2_r, dst2_r)


def _edge_body(w_ref, ef_ref, xr4_ref, w1c_ref, w1d_ref, b1_ref, g_ref,
               be_ref, wc_ref, c0_ref, nis_ref, sc_ref):
    xr4 = xr4_ref[...]
    d2 = xr4[:, 3:4]
    dist = jnp.exp(d2 * nis_ref[...])
    pre = (w_ref[...]
           + jnp.dot(ef_ref[...], w1c_ref[...], preferred_element_type=jnp.float32)
           + jnp.dot(dist, w1d_ref[...], preferred_element_type=jnp.float32)
           + b1_ref[...])
    e1 = jnp.where(pre >= 0, pre, 0.01 * pre)
    m = jnp.mean(e1, axis=-1, keepdims=True)
    cen = e1 - m
    var = jnp.mean(cen * cen, axis=-1, keepdims=True)
    e2 = g_ref[...] * cen * lax.rsqrt(var + 1e-5) + be_ref[...]
    coef = jnp.dot(e2, wc_ref[...], preferred_element_type=jnp.float32) + c0_ref[...]
    sc_ref[...] = jnp.concatenate(
        [e2, xr4[:, :3] * coef, jnp.ones_like(coef),
         jnp.zeros((e2.shape[0], 12), jnp.float32)], axis=1)


def _edge(w, ef, xr4, w1c, w1d, b1, g, be, wc, c0, nis):
    be_blk = 3200
    nf = ef.shape[1]
    return pl.pallas_call(
        _edge_body,
        grid=(E // be_blk,),
        in_specs=[
            pl.BlockSpec((be_blk, D), lambda i: (i, 0)),
            pl.BlockSpec((be_blk, nf), lambda i: (i, 0)),
            pl.BlockSpec((be_blk, 4), lambda i: (i, 0)),
            pl.BlockSpec((nf, D), lambda i: (0, 0)),
            pl.BlockSpec((15, D), lambda i: (0, 0)),
            pl.BlockSpec((1, D), lambda i: (0, 0)),
            pl.BlockSpec((1, D), lambda i: (0, 0)),
            pl.BlockSpec((1, D), lambda i: (0, 0)),
            pl.BlockSpec((D, 1), lambda i: (0, 0)),
            pl.BlockSpec((1, 1), lambda i: (0, 0)),
            pl.BlockSpec((1, 15), lambda i: (0, 0)),
        ],
        out_specs=pl.BlockSpec((be_blk, 80), lambda i: (i, 0)),
        out_shape=jax.ShapeDtypeStruct((E, 80), jnp.float32),
    )(w, ef, xr4, w1c, w1d, b1, g, be, wc, c0, nis)


def _scatter_branch(sc_hbm, dst2_hbm, acc, scb, d2d, wid):
    pltpu.sync_copy(dst2_hbm.at[wid], d2d)

    def chunk(j, carry):
        off = wid * EW + j * CH
        pltpu.sync_copy(sc_hbm.at[pl.ds(off, CH)], scb)
        pltpu.sync_copy(scb, acc.at[d2d.at[j]], add=True)
        return carry

    lax.fori_loop(0, NCHUNK, chunk, 0)


def _sc_scatter(sc_l, dst2_l, sc_r, dst2_r):
    mesh = plsc.VectorSubcoreMesh(core_axis_name="c", subcore_axis_name="s", num_cores=NC, num_subcores=NS)
    rows = N // NS
    rsub = 125

    @functools.partial(
        pl.kernel,
        out_type=[
            jax.ShapeDtypeStruct((NC, N, 80), jnp.float32),
            jax.ShapeDtypeStruct((NC, N, 80), jnp.float32),
        ],
        mesh=mesh,
        compiler_params=pltpu.CompilerParams(needs_layout_passes=False, use_tc_tiling_on_sc=False),
        scratch_types=[
            pltpu.VMEM_SHARED((N, 80), jnp.float32),
            pltpu.VMEM_SHARED((N, 80), jnp.float32),
            pltpu.VMEM((rsub, 80), jnp.float32),
            pltpu.VMEM((CH, 80), jnp.float32),
            pltpu.VMEM((NCHUNK, CH), jnp.int32),
        ],
    )
    def k(sc_l, dst2_l, sc_r, dst2_r, o_l, o_r, acc_l, acc_r, zb, scb, d2d):
        cid = lax.axis_index("c")
        sid = lax.axis_index("s")
        wid = sid * NC + cid
        z16 = jnp.zeros((LANES,), jnp.float32)

        def zf(i, c):
            r, cc = i // 5, i % 5
            zb[r, pl.ds(cc * LANES, LANES)] = z16
            return c

        lax.fori_loop(0, rsub * 5, zf, 0)

        row0 = sid * rows
        for t in range(rows // rsub):
            pltpu.sync_copy(zb, acc_l.at[pl.ds(row0 + t * rsub, rsub)])
            pltpu.sync_copy(zb, acc_r.at[pl.ds(row0 + t * rsub, rsub)])
        plsc.subcore_barrier()

        _scatter_branch(sc_l, dst2_l, acc_l, scb, d2d, wid)
        _scatter_branch(sc_r, dst2_r, acc_r, scb, d2d, wid)

        plsc.subcore_barrier()
        for t in range(rows // rsub):
            r0 = row0 + t * rsub
            pltpu.sync_copy(acc_l.at[pl.ds(r0, rsub)], o_l.at[cid, pl.ds(r0, rsub)])
            pltpu.sync_copy(acc_r.at[pl.ds(r0, rsub)], o_r.at[cid, pl.ds(r0, rsub)])

    return k(sc_l, dst2_l, sc_r, dst2_r)


def _node_body(x_ref, ox_ref, h_ref, oh_ref, a_ref, ew2_ref,
               eb2_ref, w1h_ref, w1a_ref, w1o_ref, b1_ref, g_ref, be_ref,
               w2_ref, b2_ref, xo_ref, ho_ref):
    a = a_ref[0] + a_ref[1]
    cnt = a[:, 67:68]
    c = jnp.maximum(cnt, 1.0)
    ind = jnp.minimum(cnt, 1.0)
    se2 = a[:, :D]
    aggr = (jnp.dot(se2 / c, ew2_ref[...], preferred_element_type=jnp.float32)
            + eb2_ref[...] * ind)
    sw = a[:, 64:67]
    xo_ref[...] = 0.25 * ox_ref[...] + 0.75 * x_ref[...] + sw / c
    h = h_ref[...]
    pre = (jnp.dot(h, w1h_ref[...], preferred_element_type=jnp.float32)
           + jnp.dot(aggr, w1a_ref[...], preferred_element_type=jnp.float32)
           + jnp.dot(oh_ref[...], w1o_ref[...], preferred_element_type=jnp.float32)
           + b1_ref[...])
    e1 = jnp.where(pre >= 0, pre, 0.01 * pre)
    m = jnp.mean(e1, axis=-1, keepdims=True)
    cen = e1 - m
    var = jnp.mean(cen * cen, axis=-1, keepdims=True)
    ln = g_ref[...] * cen * lax.rsqrt(var + 1e-5) + be_ref[...]
    hm = jnp.dot(ln, w2_ref[...], preferred_element_type=jnp.float32) + b2_ref[...]
    ho_ref[...] = 0.5 * hm + 0.5 * h


def _node(x, ox, h, oh, a, ew2, eb2, w1h, w1a, w1o, b1, g, be, w2, b2):
    blk = 1000
    full = lambda shp: pl.BlockSpec(shp, lambda i: tuple(0 for _ in shp))
    return pl.pallas_call(
        _node_body,
        grid=(N // blk,),
        in_specs=[
            pl.BlockSpec((blk, 3), lambda i: (i, 0)),
            pl.BlockSpec((blk, 3), lambda i: (i, 0)),
            pl.BlockSpec((blk, D), lambda i: (i, 0)),
            pl.BlockSpec((blk, D), lambda i: (i, 0)),
            pl.BlockSpec((NC, blk, 80), lambda i: (0, i, 0)),
            full((D, D)),
            full((1, D)),
            full((D, D)),
            full((D, D)),
            full((D, D)),
            full((1, D)),
            full((1, D)),
            full((1, D)),
            full((D, D)),
            full((1, D)),
        ],
        out_specs=[
            pl.BlockSpec((blk, 3), lambda i: (i, 0)),
            pl.BlockSpec((blk, D), lambda i: (i, 0)),
        ],
        out_shape=[
            jax.ShapeDtypeStruct((N, 3), jnp.float32),
            jax.ShapeDtypeStruct((N, D), jnp.float32),
        ],
    )(x, ox, h, oh, a, ew2, eb2, w1h, w1a, w1o, b1, g, be, w2, b2)


def _branch_weights(p, pre, nf):
    w1 = p[pre + '_e_W1']
    wa, wb = w1[:D], w1[D:2 * D]
    w1c = w1[2 * D:2 * D + nf]
    w1d = w1[2 * D + nf:]
    b1 = p[pre + '_e_b1'].reshape(1, D)
    g = p[pre + '_e_g'].reshape(1, D)
    be = p[pre + '_e_be'].reshape(1, D)
    wc = p[pre + '_e_W2'] @ p[pre + '_x_W']
    c0 = (p[pre + '_e_b2'] @ p[pre + '_x_W'] + p[pre + '_x_b']).reshape(1, 1)
    nis = jnp.array([[-1.0 / s for s in _SIGMAS]], jnp.float32)
    return wa, wb, w1c, w1d, b1, g, be, wc, c0, nis


def kernel(coords_lig, h_lig, orig_coords_lig, orig_h_lig, edge_feat_lig,
           coords_rec, h_rec, orig_coords_rec, orig_h_rec, edge_feat_rec,
           params, edge_index_lig, edge_index_rec):
    p = params
    wl = _branch_weights(p, 'lig', edge_feat_lig.shape[1])
    wr = _branch_weights(p, 'rec', edge_feat_rec.shape[1])

    u_l, v_l = _prep(h_lig, wl[0], wl[1])
    u_r, v_r = _prep(h_rec, wr[0], wr[1])

    src2_l = edge_index_lig[0].reshape(NW, NCHUNK, CH)
    dst2_l = edge_index_lig[1].reshape(NW, NCHUNK, CH)
    src2_r = edge_index_rec[0].reshape(NW, NCHUNK, CH)
    dst2_r = edge_index_rec[1].reshape(NW, NCHUNK, CH)
    cflat_l = coords_lig.reshape(3 * N)
    cflat_r = coords_rec.reshape(3 * N)

    w_l, xr4_l, w_r, xr4_r = _sc_gather(
        u_l, v_l, cflat_l, src2_l, dst2_l,
        u_r, v_r, cflat_r, src2_r, dst2_r)

    sc_l = _edge(w_l, edge_feat_lig, xr4_l, *wl[2:])
    sc_r = _edge(w_r, edge_feat_rec, xr4_r, *wr[2:])

    o_l, o_r = _sc_scatter(sc_l, dst2_l, sc_r, dst2_r)

    def node(pre, x, ox, h, oh, a):
        return _node(
            x, ox, h, oh, a,
            p[pre + '_e_W2'], p[pre + '_e_b2'].reshape(1, D),
            p[pre + '_n_W1'][:D], p[pre + '_n_W1'][D:2 * D],
            p[pre + '_n_W1'][2 * D:],
            p[pre + '_n_b1'].reshape(1, D), p[pre + '_n_g'].reshape(1, D),
            p[pre + '_n_be'].reshape(1, D), p[pre + '_n_W2'],
            p[pre + '_n_b2'].reshape(1, D))

    xl, hl = node('lig', coords_lig, orig_coords_lig, h_lig, orig_h_lig, o_l)
    xr, hr = node('rec', coords_rec, orig_coords_rec, h_rec, orig_h_rec, o_r)
    return jnp.concatenate([xl, hl, xr, hr], axis=1)
